# Initial kernel scaffold; baseline (speedup 1.0000x reference)
import jax
import jax.numpy as jnp
from jax.experimental import pallas as pl


def kernel(x, a_hat, w_conv, b_conv, w_res, dropout_key):
    raise NotImplementedError("write your pallas kernel here")


# speedup vs baseline: 7.2161x; 7.2161x over previous
import functools

import jax
import jax.numpy as jnp
from jax.experimental import pallas as pl
from jax.experimental.pallas import tpu as pltpu


def _round_up(a, b):
    return (a + b - 1) // b * b


def _xw_kernel(x_ref, w_ref, o_ref):
    o_ref[...] = jnp.dot(x_ref[...], w_ref[...],
                         preferred_element_type=jnp.float32).astype(o_ref.dtype)


def _agg_kernel(a_ref, xw_ref, b_ref, o_ref, *, f_pad):
    a_bf = a_ref[...].astype(jnp.bfloat16)
    acc = jnp.dot(a_bf, xw_ref[...], preferred_element_type=jnp.float32)
    h = jnp.maximum(acc[:, :f_pad] + b_ref[...], 0.0)
    o_ref[...] = (h + acc[:, f_pad:]).astype(o_ref.dtype)


@functools.partial(jax.jit, static_argnames=("tile_m",))
def _forward(x, a_hat, w_conv, b_conv, w_res, tile_m):
    n, f = x.shape
    f_pad = _round_up(f, 128)
    n_pad = _round_up(n, tile_m)

    x_pad = jnp.pad(x, ((0, n_pad - n), (0, f_pad - f)))
    wc = jnp.pad(w_conv, ((0, f_pad - f), (0, f_pad - f)))
    wr = jnp.pad(w_res, ((0, f_pad - f), (0, f_pad - f)))
    w_cat = jnp.concatenate([wc, wr], axis=1)
    b_pad = jnp.pad(b_conv.reshape(1, f), ((0, 0), (0, f_pad - f)))
    a_pad = jnp.pad(a_hat, ((0, n_pad - n), (0, n_pad - n)))

    f_out = 2 * f_pad
    xw = pl.pallas_call(
        _xw_kernel,
        out_shape=jax.ShapeDtypeStruct((n_pad, f_out), jnp.bfloat16),
        grid=(n_pad // tile_m,),
        in_specs=[pl.BlockSpec((tile_m, f_pad), lambda i: (i, 0)),
                  pl.BlockSpec((f_pad, f_out), lambda i: (0, 0))],
        out_specs=pl.BlockSpec((tile_m, f_out), lambda i: (i, 0)),
        compiler_params=pltpu.CompilerParams(
            dimension_semantics=("parallel",)),
    )(x_pad, w_cat)

    out = pl.pallas_call(
        functools.partial(_agg_kernel, f_pad=f_pad),
        out_shape=jax.ShapeDtypeStruct((n_pad, f_pad), x.dtype),
        grid=(n_pad // tile_m,),
        in_specs=[pl.BlockSpec((tile_m, n_pad), lambda i: (i, 0)),
                  pl.BlockSpec((n_pad, f_out), lambda i: (0, 0)),
                  pl.BlockSpec((1, f_pad), lambda i: (0, 0))],
        out_specs=pl.BlockSpec((tile_m, f_pad), lambda i: (i, 0)),
        compiler_params=pltpu.CompilerParams(
            dimension_semantics=("parallel",),
            vmem_limit_bytes=64 * 1024 * 1024,
        ),
    )(a_pad, xw, b_pad)
    return out[:n, :f]


def kernel(x, a_hat, w_conv, b_conv, w_res, dropout_key):
    return _forward(x, a_hat, w_conv, b_conv, w_res, tile_m=512)


# speedup vs baseline: 7.8756x; 1.0914x over previous
import functools

import jax
import jax.numpy as jnp
from jax.experimental import pallas as pl
from jax.experimental.pallas import tpu as pltpu


def _round_up(a, b):
    return (a + b - 1) // b * b


def _fused_kernel(x_ref, w_ref, a_ref, b_ref, o_ref, xw_ref, *, f_pad):
    j = pl.program_id(1)

    @pl.when(j == 0)
    def _():
        xw_ref[...] = jnp.dot(
            x_ref[...], w_ref[...],
            preferred_element_type=jnp.float32).astype(jnp.bfloat16)

    @pl.when(j > 0)
    def _():
        a_bf = a_ref[...].astype(jnp.bfloat16)
        acc = jnp.dot(a_bf, xw_ref[...], preferred_element_type=jnp.float32)
        h = jnp.maximum(acc[:, :f_pad] + b_ref[...], 0.0)
        o_ref[...] = (h + acc[:, f_pad:]).astype(o_ref.dtype)


@functools.partial(jax.jit, static_argnames=("tile_m",))
def _forward(x, a_hat, w_conv, b_conv, w_res, tile_m):
    n, f = x.shape
    f_pad = _round_up(f, 128)
    n_pad = _round_up(n, 2 * tile_m)

    x_pad = jnp.pad(x, ((0, n_pad - n), (0, f_pad - f)))
    wc = jnp.pad(w_conv, ((0, f_pad - f), (0, f_pad - f)))
    wr = jnp.pad(w_res, ((0, f_pad - f), (0, f_pad - f)))
    w_cat = jnp.concatenate([wc, wr], axis=1)
    b_pad = jnp.pad(b_conv.reshape(1, f), ((0, 0), (0, f_pad - f)))
    a_pad = jnp.pad(a_hat, ((0, n_pad - n), (0, n_pad - n)))

    f_out = 2 * f_pad
    n_i = n_pad // tile_m
    nipc = n_i // 2

    def _panel(c, j):
        return c * nipc + jnp.maximum(j - 1, 0)

    out = pl.pallas_call(
        functools.partial(_fused_kernel, f_pad=f_pad),
        out_shape=jax.ShapeDtypeStruct((n_pad, f_pad), x.dtype),
        grid=(2, nipc + 1),
        in_specs=[
            pl.BlockSpec((n_pad, f_pad), lambda c, j: (0, 0)),
            pl.BlockSpec((f_pad, f_out), lambda c, j: (0, 0)),
            pl.BlockSpec((tile_m, n_pad), lambda c, j: (_panel(c, j), 0)),
            pl.BlockSpec((1, f_pad), lambda c, j: (0, 0)),
        ],
        out_specs=pl.BlockSpec((tile_m, f_pad), lambda c, j: (_panel(c, j), 0)),
        scratch_shapes=[pltpu.VMEM((n_pad, f_out), jnp.bfloat16)],
        compiler_params=pltpu.CompilerParams(
            dimension_semantics=("parallel", "arbitrary"),
            vmem_limit_bytes=64 * 1024 * 1024,
        ),
    )(x_pad, w_cat, a_pad, b_pad)
    return out[:n, :f]


def kernel(x, a_hat, w_conv, b_conv, w_res, dropout_key):
    return _forward(x, a_hat, w_conv, b_conv, w_res, tile_m=512)
